# Initial kernel scaffold; baseline (speedup 1.0000x reference)
import jax
import jax.numpy as jnp
from jax.experimental import pallas as pl


def kernel(x, edge_index, W1, b1, g1, be1, W2, b2, g2, be2):
    raise NotImplementedError("write your pallas kernel here")


# speedup vs baseline: 30.6925x; 30.6925x over previous
import functools

import jax
import jax.numpy as jnp
from jax import lax
from jax.experimental import pallas as pl
from jax.experimental.pallas import tpu as pltpu
from jax.experimental.pallas import tpu_sc as plsc

N = 10000
E = 320000
D = 128
EPS = 1e-5

NC = 2
NS = 16
NW = NC * NS
EPW = E // NW
CHUNK = 80
NCHUNK = EPW // CHUNK
NPAD = 10240
RPT = NPAD // NS
CPT = N // NS
ZCH = 40

_mesh = plsc.VectorSubcoreMesh(core_axis_name="c", subcore_axis_name="s")


def _deg_body(dst_hbm, degp_hbm, idx_v, ones_v, zb_v, deg_sh):
    c = lax.axis_index("c")
    s = lax.axis_index("s")
    for i in range(CHUNK // 16):
        ones_v[pl.ds(16 * i, 16)] = jnp.full((16,), 1.0, jnp.float32)
    ones_v[pl.ds(CHUNK - 16, 16)] = jnp.full((16,), 1.0, jnp.float32)

    def zstore(i, carry):
        zb_v[pl.ds(16 * i, 16)] = jnp.zeros((16,), jnp.float32)
        return carry

    lax.fori_loop(0, RPT // 16, zstore, 0)
    pltpu.sync_copy(zb_v, deg_sh.at[pl.ds(s * RPT, RPT)])
    plsc.subcore_barrier()

    pltpu.sync_copy(dst_hbm.at[c, s], idx_v)

    def body(j, carry):
        pltpu.sync_copy(ones_v, deg_sh.at[idx_v.at[j, 0]], add=True)
        return carry

    lax.fori_loop(0, NCHUNK, body, 0)
    plsc.subcore_barrier()
    pltpu.sync_copy(deg_sh.at[pl.ds(s * RPT, RPT)],
                    degp_hbm.at[c, pl.ds(s * RPT, RPT)])


_deg_kernel = pl.kernel(
    _deg_body,
    out_type=jax.ShapeDtypeStruct((NC, NPAD), jnp.float32),
    mesh=_mesh,
    scratch_types=[
        pltpu.VMEM((NCHUNK, 1, CHUNK), jnp.int32),
        pltpu.VMEM((CHUNK,), jnp.float32),
        pltpu.VMEM((RPT,), jnp.float32),
        pltpu.VMEM_SHARED((NPAD,), jnp.float32),
    ],
)


def _scat_body(hs_hbm, src_hbm, dst_hbm, accp_hbm,
               sidx0, sidx1, sidx2, sidx3, didx0, didx1, didx2, didx3,
               rows0, rows1, zb, acc_sh,
               sem0, sem1, isem0, isem1, isem2, isem3):
    c = lax.axis_index("c")
    s = lax.axis_index("s")
    sidxs = [sidx0, sidx1, sidx2, sidx3]
    didxs = [didx0, didx1, didx2, didx3]
    isems = [isem0, isem1, isem2, isem3]
    rbufs = [rows0, rows1]
    rsems = [sem0, sem1]

    def zrow(r, carry):
        for j in range(D // 16):
            zb[r, pl.ds(16 * j, 16)] = jnp.zeros((16,), jnp.float32)
        return carry

    lax.fori_loop(0, ZCH, zrow, 0)

    def zcopy(k, carry):
        pltpu.sync_copy(zb, acc_sh.at[pl.ds(s * RPT + k * ZCH, ZCH)])
        return carry

    lax.fori_loop(0, RPT // ZCH, zcopy, 0)
    plsc.subcore_barrier()

    def idxstart(j, t):
        pltpu.async_copy(src_hbm.at[c, s, j, 0], sidxs[t], isems[t])
        pltpu.async_copy(dst_hbm.at[c, s, j, 0], didxs[t], isems[t])

    def iwait(j, t):
        pltpu.make_async_copy(src_hbm.at[c, s, j, 0], sidxs[t], isems[t]).wait()
        pltpu.make_async_copy(dst_hbm.at[c, s, j, 0], didxs[t], isems[t]).wait()

    def gather(t, rb, rs):
        pltpu.async_copy(hs_hbm.at[sidxs[t]], rb, rs)

    def gwait(t, rb, rs):
        pltpu.make_async_copy(hs_hbm.at[sidxs[t]], rb, rs).wait()

    def scat(t, rb):
        pltpu.sync_copy(rb, acc_sh.at[didxs[t]], add=True)

    for t in range(4):
        idxstart(t, t)
    iwait(0, 0)
    gather(0, rows0, sem0)
    iwait(1, 1)
    gather(1, rows1, sem1)

    def body(i, carry):
        j0 = 4 * i
        for t in range(4):
            tg = (t + 2) % 4
            gwait(t, rbufs[t % 2], rsems[t % 2])
            scat(t, rbufs[t % 2])
            idxstart(j0 + 4 + t, t)
            iwait(j0 + 2 + t, tg)
            gather(tg, rbufs[t % 2], rsems[t % 2])
        return carry

    nbody = (NCHUNK - 5) // 4
    lax.fori_loop(0, nbody, body, 0)
    B = 4 * nbody
    gwait(0, rows0, sem0)
    scat(0, rows0)
    idxstart(B + 4, 0)
    iwait(B + 2, 2)
    gather(2, rows0, sem0)
    gwait(1, rows1, sem1)
    scat(1, rows1)
    iwait(B + 3, 3)
    gather(3, rows1, sem1)
    gwait(2, rows0, sem0)
    scat(2, rows0)
    iwait(B + 4, 0)
    gather(0, rows0, sem0)
    gwait(3, rows1, sem1)
    scat(3, rows1)
    gwait(0, rows0, sem0)
    scat(0, rows0)

    plsc.subcore_barrier()
    pltpu.sync_copy(acc_sh.at[pl.ds(s * RPT, RPT)],
                    accp_hbm.at[c, pl.ds(s * RPT, RPT)])


_scat_kernel = pl.kernel(
    _scat_body,
    out_type=jax.ShapeDtypeStruct((NC, NPAD, D), jnp.float32),
    mesh=_mesh,
    scratch_types=[
        pltpu.VMEM((CHUNK,), jnp.int32),
        pltpu.VMEM((CHUNK,), jnp.int32),
        pltpu.VMEM((CHUNK,), jnp.int32),
        pltpu.VMEM((CHUNK,), jnp.int32),
        pltpu.VMEM((CHUNK,), jnp.int32),
        pltpu.VMEM((CHUNK,), jnp.int32),
        pltpu.VMEM((CHUNK,), jnp.int32),
        pltpu.VMEM((CHUNK,), jnp.int32),
        pltpu.VMEM((CHUNK, D), jnp.float32),
        pltpu.VMEM((CHUNK, D), jnp.float32),
        pltpu.VMEM((ZCH, D), jnp.float32),
        pltpu.VMEM_SHARED((NPAD, D), jnp.float32),
        pltpu.SemaphoreType.DMA,
        pltpu.SemaphoreType.DMA,
        pltpu.SemaphoreType.DMA,
        pltpu.SemaphoreType.DMA,
        pltpu.SemaphoreType.DMA,
        pltpu.SemaphoreType.DMA,
    ],
)


def _mm_body(x_ref, w_ref, dis_ref, h_ref, hs_ref):
    h = jnp.dot(x_ref[...], w_ref[...], preferred_element_type=jnp.float32)
    h_ref[...] = h
    hs_ref[...] = h * dis_ref[...]


_mm_kernel = pl.pallas_call(
    _mm_body,
    out_shape=[jax.ShapeDtypeStruct((N, D), jnp.float32)] * 2,
)


def _fuse_mid_body(accp_ref, h_ref, dis_ref, b_ref, g_ref, be_ref, w2_ref,
                   h2_ref, hs2_ref):
    dis = dis_ref[...]
    acc = accp_ref[0, :N] + accp_ref[1, :N]
    o = acc * dis + h_ref[...] * (dis * dis) + b_ref[...]
    m = jnp.mean(o, axis=0, keepdims=True)
    cen = o - m
    v = jnp.mean(cen * cen, axis=0, keepdims=True)
    y = g_ref[...] * (cen * lax.rsqrt(v + EPS)) + be_ref[...]
    x2 = jnp.maximum(y, 0.0)
    h2 = jnp.dot(x2, w2_ref[...], preferred_element_type=jnp.float32)
    h2_ref[...] = h2
    hs2_ref[...] = h2 * dis


_fuse_mid_kernel = pl.pallas_call(
    _fuse_mid_body,
    out_shape=[jax.ShapeDtypeStruct((N, D), jnp.float32)] * 2,
)


def _fuse_out_body(accp_ref, h_ref, dis_ref, b_ref, g_ref, be_ref, out_ref):
    dis = dis_ref[...]
    acc = accp_ref[0, :N] + accp_ref[1, :N]
    o = acc * dis + h_ref[...] * (dis * dis) + b_ref[...]
    m = jnp.mean(o, axis=0, keepdims=True)
    cen = o - m
    v = jnp.mean(cen * cen, axis=0, keepdims=True)
    out_ref[...] = g_ref[...] * (cen * lax.rsqrt(v + EPS)) + be_ref[...]


_fuse_out_kernel = pl.pallas_call(
    _fuse_out_body,
    out_shape=jax.ShapeDtypeStruct((N, D), jnp.float32),
)


@jax.jit
def kernel(x, edge_index, W1, b1, g1, be1, W2, b2, g2, be2):
    ei = edge_index.astype(jnp.int32)
    src4 = ei[0].reshape(NC, NS, NCHUNK, 1, CHUNK)
    dst4 = ei[1].reshape(NC, NS, NCHUNK, 1, CHUNK)

    degp = _deg_kernel(dst4)
    deg = degp[0, :N] + degp[1, :N] + 1.0
    dis = lax.rsqrt(deg)[:, None]

    b1r, g1r, be1r = b1[None, :], g1[None, :], be1[None, :]
    b2r, g2r, be2r = b2[None, :], g2[None, :], be2[None, :]

    h1, hs1 = _mm_kernel(x, W1, dis)
    acc1 = _scat_kernel(hs1, src4, dst4)
    h2, hs2 = _fuse_mid_kernel(acc1, h1, dis, b1r, g1r, be1r, W2)
    acc2 = _scat_kernel(hs2, src4, dst4)
    return _fuse_out_kernel(acc2, h2, dis, b2r, g2r, be2r)


# speedup vs baseline: 31.3833x; 1.0225x over previous
import functools

import jax
import jax.numpy as jnp
from jax import lax
from jax.experimental import pallas as pl
from jax.experimental.pallas import tpu as pltpu
from jax.experimental.pallas import tpu_sc as plsc

N = 10000
E = 320000
D = 128
EPS = 1e-5

NC = 2
NS = 16
NW = NC * NS
EPW = E // NW
CHUNK = 80
NCHUNK = EPW // CHUNK
NPAD = 10240
RPT = NPAD // NS
CPT = N // NS
ZCH = 40

_mesh = plsc.VectorSubcoreMesh(core_axis_name="c", subcore_axis_name="s")


def _deg_body(dst_hbm, degp_hbm, idx_v, ones_v, zb_v, deg_sh):
    c = lax.axis_index("c")
    s = lax.axis_index("s")
    for i in range(CHUNK // 16):
        ones_v[pl.ds(16 * i, 16)] = jnp.full((16,), 1.0, jnp.float32)
    ones_v[pl.ds(CHUNK - 16, 16)] = jnp.full((16,), 1.0, jnp.float32)

    def zstore(i, carry):
        zb_v[pl.ds(16 * i, 16)] = jnp.zeros((16,), jnp.float32)
        return carry

    lax.fori_loop(0, RPT // 16, zstore, 0)
    pltpu.sync_copy(zb_v, deg_sh.at[pl.ds(s * RPT, RPT)])
    plsc.subcore_barrier()

    pltpu.sync_copy(dst_hbm.at[c, s], idx_v)

    def body(j, carry):
        pltpu.sync_copy(ones_v, deg_sh.at[idx_v.at[j, 0]], add=True)
        return carry

    lax.fori_loop(0, NCHUNK, body, 0)
    plsc.subcore_barrier()
    pltpu.sync_copy(deg_sh.at[pl.ds(s * RPT, RPT)],
                    degp_hbm.at[c, pl.ds(s * RPT, RPT)])


_deg_kernel = pl.kernel(
    _deg_body,
    out_type=jax.ShapeDtypeStruct((NC, NPAD), jnp.float32),
    mesh=_mesh,
    scratch_types=[
        pltpu.VMEM((NCHUNK, 1, CHUNK), jnp.int32),
        pltpu.VMEM((CHUNK,), jnp.float32),
        pltpu.VMEM((RPT,), jnp.float32),
        pltpu.VMEM_SHARED((NPAD,), jnp.float32),
    ],
)


def _scat_body(hs_hbm, src_hbm, dst_hbm, accp_hbm,
               sidx0, sidx1, sidx2, sidx3, sidx4, sidx5, sidx6, sidx7,
               didx0, didx1, didx2, didx3, didx4, didx5, didx6, didx7,
               rows0, rows1, rows2, rows3, zb, acc_sh,
               gsem0, gsem1, gsem2, gsem3, ssem0, ssem1, ssem2, ssem3,
               isem0, isem1, isem2, isem3, isem4, isem5, isem6, isem7):
    c = lax.axis_index("c")
    s = lax.axis_index("s")
    sidxs = [sidx0, sidx1, sidx2, sidx3, sidx4, sidx5, sidx6, sidx7]
    didxs = [didx0, didx1, didx2, didx3, didx4, didx5, didx6, didx7]
    rows = [rows0, rows1, rows2, rows3]
    gsems = [gsem0, gsem1, gsem2, gsem3]
    ssems = [ssem0, ssem1, ssem2, ssem3]
    isems = [isem0, isem1, isem2, isem3, isem4, isem5, isem6, isem7]

    def zrow(r, carry):
        for j in range(D // 16):
            zb[r, pl.ds(16 * j, 16)] = jnp.zeros((16,), jnp.float32)
        return carry

    lax.fori_loop(0, ZCH, zrow, 0)

    def zcopy(k, carry):
        pltpu.sync_copy(zb, acc_sh.at[pl.ds(s * RPT + k * ZCH, ZCH)])
        return carry

    lax.fori_loop(0, RPT // ZCH, zcopy, 0)
    plsc.subcore_barrier()

    def idxstart(j, p8):
        pltpu.async_copy(src_hbm.at[c, s, j, 0], sidxs[p8], isems[p8])
        pltpu.async_copy(dst_hbm.at[c, s, j, 0], didxs[p8], isems[p8])

    def iwait(j, p8):
        pltpu.make_async_copy(src_hbm.at[c, s, j, 0], sidxs[p8], isems[p8]).wait()
        pltpu.make_async_copy(dst_hbm.at[c, s, j, 0], didxs[p8], isems[p8]).wait()

    def gather(p4, p8):
        pltpu.async_copy(hs_hbm.at[sidxs[p8]], rows[p4], gsems[p4])

    def gwait(p4, p8):
        pltpu.make_async_copy(hs_hbm.at[sidxs[p8]], rows[p4], gsems[p4]).wait()

    def sstart(p4, p8):
        pltpu.async_copy(rows[p4], acc_sh.at[didxs[p8]], ssems[p4], add=True)

    def swait(p4, p8):
        pltpu.make_async_copy(rows[p4], acc_sh.at[didxs[p8]], ssems[p4]).wait()

    def step(j, p4, p8, do_c, do_d, do_ef):
        gwait(p4, p8)
        sstart(p4, p8)
        if do_c:
            idxstart(j + 4, (p8 + 4) % 8)
        if do_d:
            swait((p4 + 2) % 4, (p8 + 2) % 8)
        if do_ef:
            iwait(j + 2, (p8 + 2) % 8)
            gather((p4 + 2) % 4, (p8 + 2) % 8)

    for j in range(4):
        idxstart(j, j)
    iwait(0, 0)
    gather(0, 0)
    iwait(1, 1)
    gather(1, 1)
    step(0, 0, 0, True, False, True)
    step(1, 1, 1, True, False, True)

    def body(i, carry):
        j0 = 8 * i + 2
        for t in range(8):
            step(j0 + t, (t + 2) % 4, (t + 2) % 8, True, True, True)
        return carry

    NTAIL = NCHUNK - 11
    nbody = (NTAIL - 2) // 8
    lax.fori_loop(0, nbody, body, 0)
    for j in range(NTAIL, NCHUNK):
        step(j, j % 4, j % 8, j + 4 < NCHUNK, True, j + 2 < NCHUNK)
    swait((NCHUNK - 2) % 4, (NCHUNK - 2) % 8)
    swait((NCHUNK - 1) % 4, (NCHUNK - 1) % 8)

    plsc.subcore_barrier()
    pltpu.sync_copy(acc_sh.at[pl.ds(s * RPT, RPT)],
                    accp_hbm.at[c, pl.ds(s * RPT, RPT)])


_scat_kernel = pl.kernel(
    _scat_body,
    out_type=jax.ShapeDtypeStruct((NC, NPAD, D), jnp.float32),
    mesh=_mesh,
    scratch_types=(
        [pltpu.VMEM((CHUNK,), jnp.int32)] * 16
        + [pltpu.VMEM((CHUNK, D), jnp.float32)] * 4
        + [pltpu.VMEM((ZCH, D), jnp.float32)]
        + [pltpu.VMEM_SHARED((NPAD, D), jnp.float32)]
        + [pltpu.SemaphoreType.DMA] * 16
    ),
)


def _mm_body(x_ref, w_ref, dis_ref, h_ref, hs_ref):
    h = jnp.dot(x_ref[...], w_ref[...], preferred_element_type=jnp.float32)
    h_ref[...] = h
    hs_ref[...] = h * dis_ref[...]


_mm_kernel = pl.pallas_call(
    _mm_body,
    out_shape=[jax.ShapeDtypeStruct((N, D), jnp.float32)] * 2,
)


def _fuse_mid_body(accp_ref, h_ref, dis_ref, b_ref, g_ref, be_ref, w2_ref,
                   h2_ref, hs2_ref):
    dis = dis_ref[...]
    acc = accp_ref[0, :N] + accp_ref[1, :N]
    o = acc * dis + h_ref[...] * (dis * dis) + b_ref[...]
    m = jnp.mean(o, axis=0, keepdims=True)
    cen = o - m
    v = jnp.mean(cen * cen, axis=0, keepdims=True)
    y = g_ref[...] * (cen * lax.rsqrt(v + EPS)) + be_ref[...]
    x2 = jnp.maximum(y, 0.0)
    h2 = jnp.dot(x2, w2_ref[...], preferred_element_type=jnp.float32)
    h2_ref[...] = h2
    hs2_ref[...] = h2 * dis


_fuse_mid_kernel = pl.pallas_call(
    _fuse_mid_body,
    out_shape=[jax.ShapeDtypeStruct((N, D), jnp.float32)] * 2,
)


def _fuse_out_body(accp_ref, h_ref, dis_ref, b_ref, g_ref, be_ref, out_ref):
    dis = dis_ref[...]
    acc = accp_ref[0, :N] + accp_ref[1, :N]
    o = acc * dis + h_ref[...] * (dis * dis) + b_ref[...]
    m = jnp.mean(o, axis=0, keepdims=True)
    cen = o - m
    v = jnp.mean(cen * cen, axis=0, keepdims=True)
    out_ref[...] = g_ref[...] * (cen * lax.rsqrt(v + EPS)) + be_ref[...]


_fuse_out_kernel = pl.pallas_call(
    _fuse_out_body,
    out_shape=jax.ShapeDtypeStruct((N, D), jnp.float32),
)


@jax.jit
def kernel(x, edge_index, W1, b1, g1, be1, W2, b2, g2, be2):
    ei = edge_index.astype(jnp.int32)
    src4 = ei[0].reshape(NC, NS, NCHUNK, 1, CHUNK)
    dst4 = ei[1].reshape(NC, NS, NCHUNK, 1, CHUNK)

    degp = _deg_kernel(dst4)
    deg = degp[0, :N] + degp[1, :N] + 1.0
    dis = lax.rsqrt(deg)[:, None]

    b1r, g1r, be1r = b1[None, :], g1[None, :], be1[None, :]
    b2r, g2r, be2r = b2[None, :], g2[None, :], be2[None, :]

    h1, hs1 = _mm_kernel(x, W1, dis)
    acc1 = _scat_kernel(hs1, src4, dst4)
    h2, hs2 = _fuse_mid_kernel(acc1, h1, dis, b1r, g1r, be1r, W2)
    acc2 = _scat_kernel(hs2, src4, dst4)
    return _fuse_out_kernel(acc2, h2, dis, b2r, g2r, be2r)


# speedup vs baseline: 31.9710x; 1.0187x over previous
import functools

import jax
import jax.numpy as jnp
from jax import lax
from jax.experimental import pallas as pl
from jax.experimental.pallas import tpu as pltpu
from jax.experimental.pallas import tpu_sc as plsc

N = 10000
E = 320000
D = 128
EPS = 1e-5

NC = 2
NS = 16
NW = NC * NS
EPW = E // NW
CHUNK = 80
NCHUNK = EPW // CHUNK
NPAD = 10240
RPT = NPAD // NS
CPT = N // NS
ZCH = 40

_mesh = plsc.VectorSubcoreMesh(core_axis_name="c", subcore_axis_name="s")


def _deg_body(dst_hbm, degp_hbm, idx_v, ones_v, zb_v, deg_sh):
    c = lax.axis_index("c")
    s = lax.axis_index("s")
    for i in range(CHUNK // 16):
        ones_v[pl.ds(16 * i, 16)] = jnp.full((16,), 1.0, jnp.float32)
    ones_v[pl.ds(CHUNK - 16, 16)] = jnp.full((16,), 1.0, jnp.float32)

    def zstore(i, carry):
        zb_v[pl.ds(16 * i, 16)] = jnp.zeros((16,), jnp.float32)
        return carry

    lax.fori_loop(0, RPT // 16, zstore, 0)
    pltpu.sync_copy(zb_v, deg_sh.at[pl.ds(s * RPT, RPT)])
    plsc.subcore_barrier()

    pltpu.sync_copy(dst_hbm.at[c, s], idx_v)

    def body(j, carry):
        pltpu.sync_copy(ones_v, deg_sh.at[idx_v.at[j, 0]], add=True)
        return carry

    lax.fori_loop(0, NCHUNK, body, 0)
    plsc.subcore_barrier()
    pltpu.sync_copy(deg_sh.at[pl.ds(s * RPT, RPT)],
                    degp_hbm.at[c, pl.ds(s * RPT, RPT)])


_deg_kernel = pl.kernel(
    _deg_body,
    out_type=jax.ShapeDtypeStruct((NC, NPAD), jnp.float32),
    mesh=_mesh,
    scratch_types=[
        pltpu.VMEM((NCHUNK, 1, CHUNK), jnp.int32),
        pltpu.VMEM((CHUNK,), jnp.float32),
        pltpu.VMEM((RPT,), jnp.float32),
        pltpu.VMEM_SHARED((NPAD,), jnp.float32),
    ],
)


def _scat_body(hs_hbm, src_hbm, dst_hbm, accp_hbm,
               sidx0, sidx1, sidx2, sidx3, sidx4, sidx5, sidx6, sidx7,
               didx0, didx1, didx2, didx3, didx4, didx5, didx6, didx7,
               rows0, rows1, rows2, rows3, zb, acc_sh,
               gsem0, gsem1, gsem2, gsem3, ssem0, ssem1, ssem2, ssem3,
               isem0, isem1, isem2, isem3, isem4, isem5, isem6, isem7):
    c = lax.axis_index("c")
    s = lax.axis_index("s")
    sidxs = [sidx0, sidx1, sidx2, sidx3, sidx4, sidx5, sidx6, sidx7]
    didxs = [didx0, didx1, didx2, didx3, didx4, didx5, didx6, didx7]
    rows = [rows0, rows1, rows2, rows3]
    gsems = [gsem0, gsem1, gsem2, gsem3]
    ssems = [ssem0, ssem1, ssem2, ssem3]
    isems = [isem0, isem1, isem2, isem3, isem4, isem5, isem6, isem7]

    def zrow(r, carry):
        for j in range(D // 16):
            zb[r, pl.ds(16 * j, 16)] = jnp.zeros((16,), jnp.float32)
        return carry

    lax.fori_loop(0, ZCH, zrow, 0)

    def zcopy(k, carry):
        pltpu.sync_copy(zb, acc_sh.at[pl.ds(s * RPT + k * ZCH, ZCH)])
        return carry

    lax.fori_loop(0, RPT // ZCH, zcopy, 0)
    plsc.subcore_barrier()

    def idxstart(j, p8):
        pltpu.async_copy(src_hbm.at[c, s, j, 0], sidxs[p8], isems[p8])
        pltpu.async_copy(dst_hbm.at[c, s, j, 0], didxs[p8], isems[p8])

    def iwait(j, p8):
        pltpu.make_async_copy(src_hbm.at[c, s, j, 0], sidxs[p8], isems[p8]).wait()
        pltpu.make_async_copy(dst_hbm.at[c, s, j, 0], didxs[p8], isems[p8]).wait()

    def gather(p4, p8):
        pltpu.async_copy(hs_hbm.at[sidxs[p8]], rows[p4], gsems[p4])

    def gwait(p4, p8):
        pltpu.make_async_copy(hs_hbm.at[sidxs[p8]], rows[p4], gsems[p4]).wait()

    def sstart(p4, p8):
        pltpu.async_copy(rows[p4], acc_sh.at[didxs[p8]], ssems[p4], add=True)

    def swait(p4, p8):
        pltpu.make_async_copy(rows[p4], acc_sh.at[didxs[p8]], ssems[p4]).wait()

    def step(j, p4, p8, do_c, do_d, do_ef):
        gwait(p4, p8)
        sstart(p4, p8)
        if do_c:
            idxstart(j + 4, (p8 + 4) % 8)
        if do_d:
            swait((p4 + 2) % 4, (p8 + 2) % 8)
        if do_ef:
            iwait(j + 2, (p8 + 2) % 8)
            gather((p4 + 2) % 4, (p8 + 2) % 8)

    for j in range(4):
        idxstart(j, j)
    iwait(0, 0)
    gather(0, 0)
    iwait(1, 1)
    gather(1, 1)
    step(0, 0, 0, True, False, True)
    step(1, 1, 1, True, False, True)

    def body(i, carry):
        j0 = 8 * i + 2
        for t in range(8):
            step(j0 + t, (t + 2) % 4, (t + 2) % 8, True, True, True)
        return carry

    NTAIL = NCHUNK - 11
    nbody = (NTAIL - 2) // 8
    lax.fori_loop(0, nbody, body, 0)
    for j in range(NTAIL, NCHUNK):
        step(j, j % 4, j % 8, j + 4 < NCHUNK, True, j + 2 < NCHUNK)
    swait((NCHUNK - 2) % 4, (NCHUNK - 2) % 8)
    swait((NCHUNK - 1) % 4, (NCHUNK - 1) % 8)

    plsc.subcore_barrier()
    pltpu.sync_copy(acc_sh.at[pl.ds(s * RPT, RPT)],
                    accp_hbm.at[c, pl.ds(s * RPT, RPT)])


_scat_kernel = pl.kernel(
    _scat_body,
    out_type=jax.ShapeDtypeStruct((NC, NPAD, D), jnp.float32),
    mesh=_mesh,
    scratch_types=(
        [pltpu.VMEM((CHUNK,), jnp.int32)] * 16
        + [pltpu.VMEM((CHUNK, D), jnp.float32)] * 4
        + [pltpu.VMEM((ZCH, D), jnp.float32)]
        + [pltpu.VMEM_SHARED((NPAD, D), jnp.float32)]
        + [pltpu.SemaphoreType.DMA] * 16
    ),
)


def _mm_body(x_ref, w_ref, dis_ref, hs_ref):
    h = jnp.dot(x_ref[...], w_ref[...], preferred_element_type=jnp.float32)
    hs_ref[...] = h * dis_ref[...]


_mm_kernel = pl.pallas_call(
    _mm_body,
    out_shape=jax.ShapeDtypeStruct((N, D), jnp.float32),
)


def _fuse_mid_body(accp_ref, hs_ref, dis_ref, b_ref, g_ref, be_ref, w2_ref,
                   hs2_ref):
    dis = dis_ref[...]
    acc = accp_ref[0, :N] + accp_ref[1, :N]
    o = (acc + hs_ref[...] * dis) * dis + b_ref[...]
    m = jnp.mean(o, axis=0, keepdims=True)
    cen = o - m
    v = jnp.mean(cen * cen, axis=0, keepdims=True)
    y = g_ref[...] * (cen * lax.rsqrt(v + EPS)) + be_ref[...]
    x2 = jnp.maximum(y, 0.0)
    h2 = jnp.dot(x2, w2_ref[...], preferred_element_type=jnp.float32)
    hs2_ref[...] = h2 * dis


_fuse_mid_kernel = pl.pallas_call(
    _fuse_mid_body,
    out_shape=jax.ShapeDtypeStruct((N, D), jnp.float32),
)


def _fuse_out_body(accp_ref, hs_ref, dis_ref, b_ref, g_ref, be_ref, out_ref):
    dis = dis_ref[...]
    acc = accp_ref[0, :N] + accp_ref[1, :N]
    o = (acc + hs_ref[...] * dis) * dis + b_ref[...]
    m = jnp.mean(o, axis=0, keepdims=True)
    cen = o - m
    v = jnp.mean(cen * cen, axis=0, keepdims=True)
    out_ref[...] = g_ref[...] * (cen * lax.rsqrt(v + EPS)) + be_ref[...]


_fuse_out_kernel = pl.pallas_call(
    _fuse_out_body,
    out_shape=jax.ShapeDtypeStruct((N, D), jnp.float32),
)


@jax.jit
def kernel(x, edge_index, W1, b1, g1, be1, W2, b2, g2, be2):
    ei = edge_index.astype(jnp.int32)
    src4 = ei[0].reshape(NC, NS, NCHUNK, 1, CHUNK)
    dst4 = ei[1].reshape(NC, NS, NCHUNK, 1, CHUNK)

    degp = _deg_kernel(dst4)
    deg = degp[0, :N] + degp[1, :N] + 1.0
    dis = lax.rsqrt(deg)[:, None]

    b1r, g1r, be1r = b1[None, :], g1[None, :], be1[None, :]
    b2r, g2r, be2r = b2[None, :], g2[None, :], be2[None, :]

    hs1 = _mm_kernel(x, W1, dis)
    acc1 = _scat_kernel(hs1, src4, dst4)
    hs2 = _fuse_mid_kernel(acc1, hs1, dis, b1r, g1r, be1r, W2)
    acc2 = _scat_kernel(hs2, src4, dst4)
    return _fuse_out_kernel(acc2, hs2, dis, b2r, g2r, be2r)


# speedup vs baseline: 31.9800x; 1.0003x over previous
import functools

import jax
import jax.numpy as jnp
from jax import lax
from jax.experimental import pallas as pl
from jax.experimental.pallas import tpu as pltpu
from jax.experimental.pallas import tpu_sc as plsc

N = 10000
E = 320000
D = 128
EPS = 1e-5

NC = 2
NS = 16
NW = NC * NS
EPW = E // NW
CHUNK = 80
NCHUNK = EPW // CHUNK
NPAD = 10240
RPT = NPAD // NS
CPT = N // NS
ZCH = 40

_mesh = plsc.VectorSubcoreMesh(core_axis_name="c", subcore_axis_name="s")


def _deg_body(dst_hbm, degp_hbm, idx_v, ones_v, zb_v, deg_sh):
    c = lax.axis_index("c")
    s = lax.axis_index("s")
    for i in range(CHUNK // 16):
        ones_v[pl.ds(16 * i, 16)] = jnp.full((16,), 1.0, jnp.float32)
    ones_v[pl.ds(CHUNK - 16, 16)] = jnp.full((16,), 1.0, jnp.float32)

    def zstore(i, carry):
        zb_v[pl.ds(16 * i, 16)] = jnp.zeros((16,), jnp.float32)
        return carry

    lax.fori_loop(0, RPT // 16, zstore, 0)
    pltpu.sync_copy(zb_v, deg_sh.at[pl.ds(s * RPT, RPT)])
    plsc.subcore_barrier()

    pltpu.sync_copy(dst_hbm.at[c, s], idx_v)

    def body(j, carry):
        pltpu.sync_copy(ones_v, deg_sh.at[idx_v.at[j, 0]], add=True)
        return carry

    lax.fori_loop(0, NCHUNK, body, 0)
    plsc.subcore_barrier()
    pltpu.sync_copy(deg_sh.at[pl.ds(s * RPT, RPT)],
                    degp_hbm.at[c, pl.ds(s * RPT, RPT)])


_deg_kernel = pl.kernel(
    _deg_body,
    out_type=jax.ShapeDtypeStruct((NC, NPAD), jnp.float32),
    mesh=_mesh,
    scratch_types=[
        pltpu.VMEM((NCHUNK, 1, CHUNK), jnp.int32),
        pltpu.VMEM((CHUNK,), jnp.float32),
        pltpu.VMEM((RPT,), jnp.float32),
        pltpu.VMEM_SHARED((NPAD,), jnp.float32),
    ],
)


def _scat_body(hs_hbm, src_hbm, dst_hbm, accp_hbm,
               sidx0, sidx1, sidx2, sidx3, sidx4, sidx5, sidx6, sidx7,
               didx0, didx1, didx2, didx3, didx4, didx5, didx6, didx7,
               rows0, rows1, rows2, rows3, zb, acc_sh,
               gsem0, gsem1, gsem2, gsem3, ssem0, ssem1, ssem2, ssem3,
               isem0, isem1, isem2, isem3, isem4, isem5, isem6, isem7):
    c = lax.axis_index("c")
    s = lax.axis_index("s")
    sidxs = [sidx0, sidx1, sidx2, sidx3, sidx4, sidx5, sidx6, sidx7]
    didxs = [didx0, didx1, didx2, didx3, didx4, didx5, didx6, didx7]
    rows = [rows0, rows1, rows2, rows3]
    gsems = [gsem0, gsem1, gsem2, gsem3]
    ssems = [ssem0, ssem1, ssem2, ssem3]
    isems = [isem0, isem1, isem2, isem3, isem4, isem5, isem6, isem7]

    def zrow(r, carry):
        for j in range(D // 16):
            zb[r, pl.ds(16 * j, 16)] = jnp.zeros((16,), jnp.float32)
        return carry

    lax.fori_loop(0, ZCH, zrow, 0)

    def zcopy(k, carry):
        pltpu.sync_copy(zb, acc_sh.at[pl.ds(s * RPT + k * ZCH, ZCH)])
        return carry

    lax.fori_loop(0, RPT // ZCH, zcopy, 0)
    plsc.subcore_barrier()

    def idxstart(j, p8):
        pltpu.async_copy(src_hbm.at[c, s, j, 0], sidxs[p8], isems[p8])
        pltpu.async_copy(dst_hbm.at[c, s, j, 0], didxs[p8], isems[p8])

    def iwait(j, p8):
        pltpu.make_async_copy(src_hbm.at[c, s, j, 0], sidxs[p8], isems[p8]).wait()
        pltpu.make_async_copy(dst_hbm.at[c, s, j, 0], didxs[p8], isems[p8]).wait()

    def gather(p4, p8):
        pltpu.async_copy(hs_hbm.at[sidxs[p8]], rows[p4], gsems[p4])

    def gwait(p4, p8):
        pltpu.make_async_copy(hs_hbm.at[sidxs[p8]], rows[p4], gsems[p4]).wait()

    def sstart(p4, p8):
        pltpu.async_copy(rows[p4], acc_sh.at[didxs[p8]], ssems[p4], add=True)

    def swait(p4, p8):
        pltpu.make_async_copy(rows[p4], acc_sh.at[didxs[p8]], ssems[p4]).wait()

    def step(j, p4, p8, do_c, do_d, do_ef):
        gwait(p4, p8)
        sstart(p4, p8)
        if do_c:
            idxstart(j + 4, (p8 + 4) % 8)
        if do_d:
            swait((p4 + 2) % 4, (p8 + 2) % 8)
        if do_ef:
            iwait(j + 2, (p8 + 2) % 8)
            gather((p4 + 2) % 4, (p8 + 2) % 8)

    for j in range(4):
        idxstart(j, j)
    iwait(0, 0)
    gather(0, 0)
    iwait(1, 1)
    gather(1, 1)
    step(0, 0, 0, True, False, True)
    step(1, 1, 1, True, False, True)

    def body(i, carry):
        j0 = 8 * i + 2
        for t in range(8):
            step(j0 + t, (t + 2) % 4, (t + 2) % 8, True, True, True)
        return carry

    NTAIL = NCHUNK - 11
    nbody = (NTAIL - 2) // 8
    lax.fori_loop(0, nbody, body, 0)
    for j in range(NTAIL, NCHUNK):
        step(j, j % 4, j % 8, j + 4 < NCHUNK, True, j + 2 < NCHUNK)
    swait((NCHUNK - 2) % 4, (NCHUNK - 2) % 8)
    swait((NCHUNK - 1) % 4, (NCHUNK - 1) % 8)

    plsc.subcore_barrier()
    pltpu.sync_copy(acc_sh.at[pl.ds(s * RPT, RPT)],
                    accp_hbm.at[c, pl.ds(s * RPT, RPT)])


_scat_kernel = pl.kernel(
    _scat_body,
    out_type=jax.ShapeDtypeStruct((NC, NPAD, D), jnp.float32),
    mesh=_mesh,
    scratch_types=(
        [pltpu.VMEM((CHUNK,), jnp.int32)] * 16
        + [pltpu.VMEM((CHUNK, D), jnp.float32)] * 4
        + [pltpu.VMEM((ZCH, D), jnp.float32)]
        + [pltpu.VMEM_SHARED((NPAD, D), jnp.float32)]
        + [pltpu.SemaphoreType.DMA] * 16
    ),
)


def _mm_body(x_ref, w_ref, dis_ref, hs_ref):
    h = jnp.dot(x_ref[...], w_ref[...], preferred_element_type=jnp.float32)
    hs_ref[...] = h * dis_ref[...]


_mm_kernel = pl.pallas_call(
    _mm_body,
    out_shape=jax.ShapeDtypeStruct((N, D), jnp.float32),
)


def _fuse_mid_body(accp_ref, hs_ref, dis_ref, b_ref, g_ref, be_ref, w2_ref,
                   hs2_ref):
    dis = dis_ref[...]
    acc = accp_ref[0, :N] + accp_ref[1, :N]
    o = (acc + hs_ref[...]) * dis + b_ref[...]
    m = jnp.mean(o, axis=0, keepdims=True)
    cen = o - m
    v = jnp.mean(cen * cen, axis=0, keepdims=True)
    y = g_ref[...] * (cen * lax.rsqrt(v + EPS)) + be_ref[...]
    x2 = jnp.maximum(y, 0.0)
    h2 = jnp.dot(x2, w2_ref[...], preferred_element_type=jnp.float32)
    hs2_ref[...] = h2 * dis


_fuse_mid_kernel = pl.pallas_call(
    _fuse_mid_body,
    out_shape=jax.ShapeDtypeStruct((N, D), jnp.float32),
)


def _fuse_out_body(accp_ref, hs_ref, dis_ref, b_ref, g_ref, be_ref, out_ref):
    dis = dis_ref[...]
    acc = accp_ref[0, :N] + accp_ref[1, :N]
    o = (acc + hs_ref[...]) * dis + b_ref[...]
    m = jnp.mean(o, axis=0, keepdims=True)
    cen = o - m
    v = jnp.mean(cen * cen, axis=0, keepdims=True)
    out_ref[...] = g_ref[...] * (cen * lax.rsqrt(v + EPS)) + be_ref[...]


_fuse_out_kernel = pl.pallas_call(
    _fuse_out_body,
    out_shape=jax.ShapeDtypeStruct((N, D), jnp.float32),
)


@jax.jit
def kernel(x, edge_index, W1, b1, g1, be1, W2, b2, g2, be2):
    ei = edge_index.astype(jnp.int32)
    src4 = ei[0].reshape(NC, NS, NCHUNK, 1, CHUNK)
    dst4 = ei[1].reshape(NC, NS, NCHUNK, 1, CHUNK)

    degp = _deg_kernel(dst4)
    deg = degp[0, :N] + degp[1, :N] + 1.0
    dis = lax.rsqrt(deg)[:, None]

    b1r, g1r, be1r = b1[None, :], g1[None, :], be1[None, :]
    b2r, g2r, be2r = b2[None, :], g2[None, :], be2[None, :]

    hs1 = _mm_kernel(x, W1, dis)
    acc1 = _scat_kernel(hs1, src4, dst4)
    hs2 = _fuse_mid_kernel(acc1, hs1, dis, b1r, g1r, be1r, W2)
    acc2 = _scat_kernel(hs2, src4, dst4)
    return _fuse_out_kernel(acc2, hs2, dis, b2r, g2r, be2r)


# speedup vs baseline: 32.3720x; 1.0123x over previous
import functools

import jax
import jax.numpy as jnp
from jax import lax
from jax.experimental import pallas as pl
from jax.experimental.pallas import tpu as pltpu
from jax.experimental.pallas import tpu_sc as plsc

N = 10000
E = 320000
D = 128
EPS = 1e-5

NC = 2
NS = 16
NW = NC * NS
EPW = E // NW
CHUNK = 80
NCHUNK = EPW // CHUNK
NPAD = 10240
RPT = NPAD // NS
CPT = N // NS
ZCH = 40

_mesh = plsc.VectorSubcoreMesh(core_axis_name="c", subcore_axis_name="s")


def _deg_body(dst_hbm, degp_hbm, idx_v, ones_v, zb_v, deg_sh):
    c = lax.axis_index("c")
    s = lax.axis_index("s")
    for i in range(CHUNK // 16):
        ones_v[pl.ds(16 * i, 16)] = jnp.full((16,), 1.0, jnp.float32)
    ones_v[pl.ds(CHUNK - 16, 16)] = jnp.full((16,), 1.0, jnp.float32)

    def zstore(i, carry):
        zb_v[pl.ds(16 * i, 16)] = jnp.zeros((16,), jnp.float32)
        return carry

    lax.fori_loop(0, RPT // 16, zstore, 0)
    pltpu.sync_copy(zb_v, deg_sh.at[pl.ds(s * RPT, RPT)])
    plsc.subcore_barrier()

    pltpu.sync_copy(dst_hbm.at[c, s], idx_v)

    def body(j, carry):
        pltpu.sync_copy(ones_v, deg_sh.at[idx_v.at[j, 0]], add=True)
        return carry

    lax.fori_loop(0, NCHUNK, body, 0)
    plsc.subcore_barrier()
    pltpu.sync_copy(deg_sh.at[pl.ds(s * RPT, RPT)],
                    degp_hbm.at[c, pl.ds(s * RPT, RPT)])


_deg_kernel = pl.kernel(
    _deg_body,
    out_type=jax.ShapeDtypeStruct((NC, NPAD), jnp.float32),
    mesh=_mesh,
    scratch_types=[
        pltpu.VMEM((NCHUNK, 1, CHUNK), jnp.int32),
        pltpu.VMEM((CHUNK,), jnp.float32),
        pltpu.VMEM((RPT,), jnp.float32),
        pltpu.VMEM_SHARED((NPAD,), jnp.float32),
    ],
)


def _scat_body(hs_hbm, src_hbm, dst_hbm, accp_hbm,
               sidx0, sidx1, sidx2, sidx3, sidx4, sidx5, sidx6, sidx7,
               didx0, didx1, didx2, didx3, didx4, didx5, didx6, didx7,
               rows0, rows1, rows2, rows3, zb, acc_sh,
               gsem0, gsem1, gsem2, gsem3, ssem0, ssem1, ssem2, ssem3,
               isem0, isem1, isem2, isem3, isem4, isem5, isem6, isem7):
    c = lax.axis_index("c")
    s = lax.axis_index("s")
    sidxs = [sidx0, sidx1, sidx2, sidx3, sidx4, sidx5, sidx6, sidx7]
    didxs = [didx0, didx1, didx2, didx3, didx4, didx5, didx6, didx7]
    rows = [rows0, rows1, rows2, rows3]
    gsems = [gsem0, gsem1, gsem2, gsem3]
    ssems = [ssem0, ssem1, ssem2, ssem3]
    isems = [isem0, isem1, isem2, isem3, isem4, isem5, isem6, isem7]

    def zrow(r, carry):
        for j in range(D // 16):
            zb[r, pl.ds(16 * j, 16)] = jnp.zeros((16,), jnp.float32)
        return carry

    lax.fori_loop(0, ZCH, zrow, 0)

    def zcopy(k, carry):
        pltpu.sync_copy(zb, acc_sh.at[pl.ds(s * RPT + k * ZCH, ZCH)])
        return carry

    lax.fori_loop(0, RPT // ZCH, zcopy, 0)
    plsc.subcore_barrier()

    def idxstart(j, p8):
        pltpu.async_copy(src_hbm.at[c, s, j, 0], sidxs[p8], isems[p8])
        pltpu.async_copy(dst_hbm.at[c, s, j, 0], didxs[p8], isems[p8])

    def iwait(j, p8):
        pltpu.make_async_copy(src_hbm.at[c, s, j, 0], sidxs[p8], isems[p8]).wait()
        pltpu.make_async_copy(dst_hbm.at[c, s, j, 0], didxs[p8], isems[p8]).wait()

    def gather(p4, p8):
        pltpu.async_copy(hs_hbm.at[sidxs[p8]], rows[p4], gsems[p4])

    def gwait(p4, p8):
        pltpu.make_async_copy(hs_hbm.at[sidxs[p8]], rows[p4], gsems[p4]).wait()

    def sstart(p4, p8):
        pltpu.async_copy(rows[p4], acc_sh.at[didxs[p8]], ssems[p4], add=True)

    def swait(p4, p8):
        pltpu.make_async_copy(rows[p4], acc_sh.at[didxs[p8]], ssems[p4]).wait()

    def step(j, p4, p8, do_c, do_d, do_ef):
        gwait(p4, p8)
        sstart(p4, p8)
        if do_c:
            idxstart(j + 4, (p8 + 4) % 8)
        if do_d:
            swait((p4 + 2) % 4, (p8 + 2) % 8)
        if do_ef:
            iwait(j + 2, (p8 + 2) % 8)
            gather((p4 + 2) % 4, (p8 + 2) % 8)

    for j in range(4):
        idxstart(j, j)
    iwait(0, 0)
    gather(0, 0)
    iwait(1, 1)
    gather(1, 1)
    step(0, 0, 0, True, False, True)
    step(1, 1, 1, True, False, True)

    def body(i, carry):
        j0 = 8 * i + 2
        for t in range(8):
            step(j0 + t, (t + 2) % 4, (t + 2) % 8, True, True, True)
        return carry

    NTAIL = NCHUNK - 11
    nbody = (NTAIL - 2) // 8
    lax.fori_loop(0, nbody, body, 0)
    for j in range(NTAIL, NCHUNK):
        step(j, j % 4, j % 8, j + 4 < NCHUNK, True, j + 2 < NCHUNK)
    swait((NCHUNK - 2) % 4, (NCHUNK - 2) % 8)
    swait((NCHUNK - 1) % 4, (NCHUNK - 1) % 8)

    plsc.subcore_barrier()
    pltpu.sync_copy(acc_sh.at[pl.ds(s * RPT, RPT)],
                    accp_hbm.at[c, pl.ds(s * RPT, RPT)])


_scat_kernel = pl.kernel(
    _scat_body,
    out_type=jax.ShapeDtypeStruct((NC, NPAD, D), jnp.float32),
    mesh=_mesh,
    scratch_types=(
        [pltpu.VMEM((CHUNK,), jnp.int32)] * 16
        + [pltpu.VMEM((CHUNK, D), jnp.float32)] * 4
        + [pltpu.VMEM((ZCH, D), jnp.float32)]
        + [pltpu.VMEM_SHARED((NPAD, D), jnp.float32)]
        + [pltpu.SemaphoreType.DMA] * 16
    ),
)


def _mm_body(x_ref, w_ref, degp_ref, hs_ref, dis_ref):
    deg = degp_ref[0] + degp_ref[1] + 1.0
    dis = lax.rsqrt(deg)[:N, None]
    h = jnp.dot(x_ref[...], w_ref[...], preferred_element_type=jnp.float32)
    hs_ref[...] = h * dis
    dis_ref[...] = dis


_mm_kernel = pl.pallas_call(
    _mm_body,
    out_shape=[jax.ShapeDtypeStruct((N, D), jnp.float32),
               jax.ShapeDtypeStruct((N, 1), jnp.float32)],
)


def _fuse_mid_body(accp_ref, hs_ref, dis_ref, b_ref, g_ref, be_ref, w2_ref,
                   hs2_ref):
    dis = dis_ref[...]
    acc = accp_ref[0, :N] + accp_ref[1, :N]
    o = (acc + hs_ref[...]) * dis + b_ref[...]
    m = jnp.mean(o, axis=0, keepdims=True)
    cen = o - m
    v = jnp.mean(cen * cen, axis=0, keepdims=True)
    y = g_ref[...] * (cen * lax.rsqrt(v + EPS)) + be_ref[...]
    x2 = jnp.maximum(y, 0.0)
    h2 = jnp.dot(x2, w2_ref[...], preferred_element_type=jnp.float32)
    hs2_ref[...] = h2 * dis


_fuse_mid_kernel = pl.pallas_call(
    _fuse_mid_body,
    out_shape=jax.ShapeDtypeStruct((N, D), jnp.float32),
)


def _fuse_out_body(accp_ref, hs_ref, dis_ref, b_ref, g_ref, be_ref, out_ref):
    dis = dis_ref[...]
    acc = accp_ref[0, :N] + accp_ref[1, :N]
    o = (acc + hs_ref[...]) * dis + b_ref[...]
    m = jnp.mean(o, axis=0, keepdims=True)
    cen = o - m
    v = jnp.mean(cen * cen, axis=0, keepdims=True)
    out_ref[...] = g_ref[...] * (cen * lax.rsqrt(v + EPS)) + be_ref[...]


_fuse_out_kernel = pl.pallas_call(
    _fuse_out_body,
    out_shape=jax.ShapeDtypeStruct((N, D), jnp.float32),
)


@jax.jit
def kernel(x, edge_index, W1, b1, g1, be1, W2, b2, g2, be2):
    ei = edge_index.astype(jnp.int32)
    src4 = ei[0].reshape(NC, NS, NCHUNK, 1, CHUNK)
    dst4 = ei[1].reshape(NC, NS, NCHUNK, 1, CHUNK)

    degp = _deg_kernel(dst4)

    b1r, g1r, be1r = b1[None, :], g1[None, :], be1[None, :]
    b2r, g2r, be2r = b2[None, :], g2[None, :], be2[None, :]

    hs1, dis = _mm_kernel(x, W1, degp)
    acc1 = _scat_kernel(hs1, src4, dst4)
    hs2 = _fuse_mid_kernel(acc1, hs1, dis, b1r, g1r, be1r, W2)
    acc2 = _scat_kernel(hs2, src4, dst4)
    return _fuse_out_kernel(acc2, hs2, dis, b2r, g2r, be2r)


# speedup vs baseline: 36.4374x; 1.1256x over previous
import functools

import jax
import jax.numpy as jnp
from jax import lax
from jax.experimental import pallas as pl
from jax.experimental.pallas import tpu as pltpu
from jax.experimental.pallas import tpu_sc as plsc

N = 10000
E = 320000
D = 128
EPS = 1e-5

NC = 2
NS = 16
NW = NC * NS
EPW = E // NW
CHUNK = 80
NCHUNK = EPW // CHUNK
NPAD = 10240
RPT = NPAD // NS
CPT = N // NS
ZCH = 40

_mesh = plsc.VectorSubcoreMesh(core_axis_name="c", subcore_axis_name="s")


def _deg_body(dst_hbm, degp_hbm, idx_v, ones_v, zb_v, deg_sh, dsem):
    c = lax.axis_index("c")
    s = lax.axis_index("s")
    for i in range(CHUNK // 16):
        ones_v[pl.ds(16 * i, 16)] = jnp.full((16,), 1.0, jnp.float32)
    ones_v[pl.ds(CHUNK - 16, 16)] = jnp.full((16,), 1.0, jnp.float32)

    def zstore(i, carry):
        zb_v[pl.ds(16 * i, 16)] = jnp.zeros((16,), jnp.float32)
        return carry

    lax.fori_loop(0, RPT // 16, zstore, 0)
    pltpu.sync_copy(zb_v, deg_sh.at[pl.ds(s * RPT, RPT)])
    plsc.subcore_barrier()

    pltpu.sync_copy(dst_hbm.at[c, s], idx_v)

    def body(i, carry):
        for t in range(8):
            pltpu.async_copy(ones_v, deg_sh.at[idx_v.at[8 * i + t, 0]],
                             dsem, add=True)
        for t in range(8):
            pltpu.make_async_copy(ones_v, deg_sh.at[idx_v.at[8 * i + t, 0]],
                                  dsem).wait()
        return carry

    lax.fori_loop(0, NCHUNK // 8, body, 0)
    for j in range(NCHUNK - NCHUNK % 8, NCHUNK):
        pltpu.async_copy(ones_v, deg_sh.at[idx_v.at[j, 0]], dsem, add=True)
    for j in range(NCHUNK - NCHUNK % 8, NCHUNK):
        pltpu.make_async_copy(ones_v, deg_sh.at[idx_v.at[j, 0]], dsem).wait()
    plsc.subcore_barrier()
    pltpu.sync_copy(deg_sh.at[pl.ds(s * RPT, RPT)],
                    degp_hbm.at[c, pl.ds(s * RPT, RPT)])


_deg_kernel = pl.kernel(
    _deg_body,
    out_type=jax.ShapeDtypeStruct((NC, NPAD), jnp.float32),
    mesh=_mesh,
    scratch_types=[
        pltpu.VMEM((NCHUNK, 1, CHUNK), jnp.int32),
        pltpu.VMEM((CHUNK,), jnp.float32),
        pltpu.VMEM((RPT,), jnp.float32),
        pltpu.VMEM_SHARED((NPAD,), jnp.float32),
        pltpu.SemaphoreType.DMA,
    ],
)


def _scat_body(hs_hbm, src_hbm, dst_hbm, accp_hbm,
               sidx0, sidx1, sidx2, sidx3, sidx4, sidx5, sidx6, sidx7,
               didx0, didx1, didx2, didx3, didx4, didx5, didx6, didx7,
               rows0, rows1, rows2, rows3, zb, acc_sh,
               gsem0, gsem1, gsem2, gsem3, ssem0, ssem1, ssem2, ssem3,
               isem0, isem1, isem2, isem3, isem4, isem5, isem6, isem7):
    c = lax.axis_index("c")
    s = lax.axis_index("s")
    sidxs = [sidx0, sidx1, sidx2, sidx3, sidx4, sidx5, sidx6, sidx7]
    didxs = [didx0, didx1, didx2, didx3, didx4, didx5, didx6, didx7]
    rows = [rows0, rows1, rows2, rows3]
    gsems = [gsem0, gsem1, gsem2, gsem3]
    ssems = [ssem0, ssem1, ssem2, ssem3]
    isems = [isem0, isem1, isem2, isem3, isem4, isem5, isem6, isem7]

    def zrow(r, carry):
        for j in range(D // 16):
            zb[r, pl.ds(16 * j, 16)] = jnp.zeros((16,), jnp.float32)
        return carry

    lax.fori_loop(0, ZCH, zrow, 0)

    def zcopy(k, carry):
        pltpu.sync_copy(zb, acc_sh.at[pl.ds(s * RPT + k * ZCH, ZCH)])
        return carry

    lax.fori_loop(0, RPT // ZCH, zcopy, 0)
    plsc.subcore_barrier()

    def idxstart(j, p8):
        pltpu.async_copy(src_hbm.at[c, s, j, 0], sidxs[p8], isems[p8])
        pltpu.async_copy(dst_hbm.at[c, s, j, 0], didxs[p8], isems[p8])

    def iwait(j, p8):
        pltpu.make_async_copy(src_hbm.at[c, s, j, 0], sidxs[p8], isems[p8]).wait()
        pltpu.make_async_copy(dst_hbm.at[c, s, j, 0], didxs[p8], isems[p8]).wait()

    def gather(p4, p8):
        pltpu.async_copy(hs_hbm.at[sidxs[p8]], rows[p4], gsems[p4])

    def gwait(p4, p8):
        pltpu.make_async_copy(hs_hbm.at[sidxs[p8]], rows[p4], gsems[p4]).wait()

    def sstart(p4, p8):
        pltpu.async_copy(rows[p4], acc_sh.at[didxs[p8]], ssems[p4], add=True)

    def swait(p4, p8):
        pltpu.make_async_copy(rows[p4], acc_sh.at[didxs[p8]], ssems[p4]).wait()

    def step(j, p4, p8, do_c, do_d, do_ef):
        gwait(p4, p8)
        sstart(p4, p8)
        if do_c:
            idxstart(j + 4, (p8 + 4) % 8)
        if do_d:
            swait((p4 + 3) % 4, (p8 + 3) % 8)
        if do_ef:
            iwait(j + 3, (p8 + 3) % 8)
            gather((p4 + 3) % 4, (p8 + 3) % 8)

    for j in range(4):
        idxstart(j, j)
    iwait(0, 0)
    gather(0, 0)
    iwait(1, 1)
    gather(1, 1)
    iwait(2, 2)
    gather(2, 2)
    step(0, 0, 0, True, False, True)

    def body(i, carry):
        j0 = 8 * i + 1
        for t in range(8):
            step(j0 + t, (t + 1) % 4, (t + 1) % 8, True, True, True)
        return carry

    NTAIL = NCHUNK - 12
    nbody = (NTAIL - 1) // 8
    lax.fori_loop(0, nbody, body, 0)
    for j in range(NTAIL, NCHUNK):
        step(j, j % 4, j % 8, j + 4 < NCHUNK, True, j + 3 < NCHUNK)
    swait((NCHUNK - 1) % 4, (NCHUNK - 1) % 8)

    plsc.subcore_barrier()
    pltpu.sync_copy(acc_sh.at[pl.ds(s * RPT, RPT)],
                    accp_hbm.at[c, pl.ds(s * RPT, RPT)])


_scat_kernel = pl.kernel(
    _scat_body,
    out_type=jax.ShapeDtypeStruct((NC, NPAD, D), jnp.float32),
    mesh=_mesh,
    scratch_types=(
        [pltpu.VMEM((CHUNK,), jnp.int32)] * 16
        + [pltpu.VMEM((CHUNK, D), jnp.float32)] * 4
        + [pltpu.VMEM((ZCH, D), jnp.float32)]
        + [pltpu.VMEM_SHARED((NPAD, D), jnp.float32)]
        + [pltpu.SemaphoreType.DMA] * 16
    ),
)


def _mm_body(x_ref, w_ref, degp_ref, hs_ref, dis_ref):
    deg = degp_ref[0] + degp_ref[1] + 1.0
    dis = lax.rsqrt(deg)[:N, None]
    h = jnp.dot(x_ref[...], w_ref[...], preferred_element_type=jnp.float32)
    hs_ref[...] = h * dis
    dis_ref[...] = dis


_mm_kernel = pl.pallas_call(
    _mm_body,
    out_shape=[jax.ShapeDtypeStruct((N, D), jnp.float32),
               jax.ShapeDtypeStruct((N, 1), jnp.float32)],
)


def _fuse_mid_body(accp_ref, hs_ref, dis_ref, b_ref, g_ref, be_ref, w2_ref,
                   hs2_ref):
    dis = dis_ref[...]
    acc = accp_ref[0, :N] + accp_ref[1, :N]
    o = (acc + hs_ref[...]) * dis + b_ref[...]
    m = jnp.mean(o, axis=0, keepdims=True)
    cen = o - m
    v = jnp.mean(cen * cen, axis=0, keepdims=True)
    y = g_ref[...] * (cen * lax.rsqrt(v + EPS)) + be_ref[...]
    x2 = jnp.maximum(y, 0.0)
    h2 = jnp.dot(x2, w2_ref[...], preferred_element_type=jnp.float32)
    hs2_ref[...] = h2 * dis


_fuse_mid_kernel = pl.pallas_call(
    _fuse_mid_body,
    out_shape=jax.ShapeDtypeStruct((N, D), jnp.float32),
)


def _fuse_out_body(accp_ref, hs_ref, dis_ref, b_ref, g_ref, be_ref, out_ref):
    dis = dis_ref[...]
    acc = accp_ref[0, :N] + accp_ref[1, :N]
    o = (acc + hs_ref[...]) * dis + b_ref[...]
    m = jnp.mean(o, axis=0, keepdims=True)
    cen = o - m
    v = jnp.mean(cen * cen, axis=0, keepdims=True)
    out_ref[...] = g_ref[...] * (cen * lax.rsqrt(v + EPS)) + be_ref[...]


_fuse_out_kernel = pl.pallas_call(
    _fuse_out_body,
    out_shape=jax.ShapeDtypeStruct((N, D), jnp.float32),
)


@jax.jit
def kernel(x, edge_index, W1, b1, g1, be1, W2, b2, g2, be2):
    ei = edge_index.astype(jnp.int32)
    src4 = ei[0].reshape(NC, NS, NCHUNK, 1, CHUNK)
    dst4 = ei[1].reshape(NC, NS, NCHUNK, 1, CHUNK)

    degp = _deg_kernel(dst4)

    b1r, g1r, be1r = b1[None, :], g1[None, :], be1[None, :]
    b2r, g2r, be2r = b2[None, :], g2[None, :], be2[None, :]

    hs1, dis = _mm_kernel(x, W1, degp)
    acc1 = _scat_kernel(hs1, src4, dst4)
    hs2 = _fuse_mid_kernel(acc1, hs1, dis, b1r, g1r, be1r, W2)
    acc2 = _scat_kernel(hs2, src4, dst4)
    return _fuse_out_kernel(acc2, hs2, dis, b2r, g2r, be2r)


# speedup vs baseline: 36.9373x; 1.0137x over previous
import functools

import jax
import jax.numpy as jnp
from jax import lax
from jax.experimental import pallas as pl
from jax.experimental.pallas import tpu as pltpu
from jax.experimental.pallas import tpu_sc as plsc

N = 10000
E = 320000
D = 128
EPS = 1e-5

NC = 2
NS = 16
NW = NC * NS
EPW = E // NW
CHUNK = 80
NCHUNK = EPW // CHUNK
NPAD = 10240
RPT = NPAD // NS
CPT = N // NS
ZCH = 40

_mesh = plsc.VectorSubcoreMesh(core_axis_name="c", subcore_axis_name="s")


def _deg_body(dst_hbm, degp_hbm, idx_v, ones_v, zb_v, deg_sh, dsem):
    c = lax.axis_index("c")
    s = lax.axis_index("s")
    for i in range(CHUNK // 16):
        ones_v[pl.ds(16 * i, 16)] = jnp.full((16,), 1.0, jnp.float32)
    ones_v[pl.ds(CHUNK - 16, 16)] = jnp.full((16,), 1.0, jnp.float32)

    def zstore(i, carry):
        zb_v[pl.ds(16 * i, 16)] = jnp.zeros((16,), jnp.float32)
        return carry

    lax.fori_loop(0, RPT // 16, zstore, 0)
    pltpu.sync_copy(zb_v, deg_sh.at[pl.ds(s * RPT, RPT)])
    plsc.subcore_barrier()

    pltpu.sync_copy(dst_hbm.at[c * NS + s], idx_v)

    def body(i, carry):
        for t in range(8):
            pltpu.async_copy(ones_v, deg_sh.at[idx_v.at[8 * i + t]],
                             dsem, add=True)
        for t in range(8):
            pltpu.make_async_copy(ones_v, deg_sh.at[idx_v.at[8 * i + t]],
                                  dsem).wait()
        return carry

    lax.fori_loop(0, NCHUNK // 8, body, 0)
    for j in range(NCHUNK - NCHUNK % 8, NCHUNK):
        pltpu.async_copy(ones_v, deg_sh.at[idx_v.at[j]], dsem, add=True)
    for j in range(NCHUNK - NCHUNK % 8, NCHUNK):
        pltpu.make_async_copy(ones_v, deg_sh.at[idx_v.at[j]], dsem).wait()
    plsc.subcore_barrier()
    pltpu.sync_copy(deg_sh.at[pl.ds(s * RPT, RPT)],
                    degp_hbm.at[c, pl.ds(s * RPT, RPT)])


_deg_kernel = pl.kernel(
    _deg_body,
    out_type=jax.ShapeDtypeStruct((NC, NPAD), jnp.float32),
    mesh=_mesh,
    scratch_types=[
        pltpu.VMEM((NCHUNK, CHUNK), jnp.int32),
        pltpu.VMEM((CHUNK,), jnp.float32),
        pltpu.VMEM((RPT,), jnp.float32),
        pltpu.VMEM_SHARED((NPAD,), jnp.float32),
        pltpu.SemaphoreType.DMA,
    ],
)


def _scat_body(hs_hbm, src_hbm, dst_hbm, accp_hbm,
               sidx0, sidx1, sidx2, sidx3, sidx4, sidx5, sidx6, sidx7,
               didx0, didx1, didx2, didx3, didx4, didx5, didx6, didx7,
               rows0, rows1, rows2, rows3, zb, acc_sh,
               gsem0, gsem1, gsem2, gsem3, ssem0, ssem1, ssem2, ssem3,
               isem0, isem1, isem2, isem3, isem4, isem5, isem6, isem7):
    c = lax.axis_index("c")
    s = lax.axis_index("s")
    sidxs = [sidx0, sidx1, sidx2, sidx3, sidx4, sidx5, sidx6, sidx7]
    didxs = [didx0, didx1, didx2, didx3, didx4, didx5, didx6, didx7]
    rows = [rows0, rows1, rows2, rows3]
    gsems = [gsem0, gsem1, gsem2, gsem3]
    ssems = [ssem0, ssem1, ssem2, ssem3]
    isems = [isem0, isem1, isem2, isem3, isem4, isem5, isem6, isem7]

    def zrow(r, carry):
        for j in range(D // 16):
            zb[r, pl.ds(16 * j, 16)] = jnp.zeros((16,), jnp.float32)
        return carry

    lax.fori_loop(0, ZCH, zrow, 0)

    def zcopy(k, carry):
        pltpu.sync_copy(zb, acc_sh.at[pl.ds(s * RPT + k * ZCH, ZCH)])
        return carry

    lax.fori_loop(0, RPT // ZCH, zcopy, 0)
    plsc.subcore_barrier()

    base = (c * NS + s) * EPW

    def idxstart(j, p8):
        pltpu.async_copy(src_hbm.at[pl.ds(base + j * CHUNK, CHUNK)],
                         sidxs[p8], isems[p8])
        pltpu.async_copy(dst_hbm.at[pl.ds(base + j * CHUNK, CHUNK)],
                         didxs[p8], isems[p8])

    def iwait(j, p8):
        pltpu.make_async_copy(src_hbm.at[pl.ds(base + j * CHUNK, CHUNK)],
                              sidxs[p8], isems[p8]).wait()
        pltpu.make_async_copy(dst_hbm.at[pl.ds(base + j * CHUNK, CHUNK)],
                              didxs[p8], isems[p8]).wait()

    def gather(p4, p8):
        pltpu.async_copy(hs_hbm.at[sidxs[p8]], rows[p4], gsems[p4])

    def gwait(p4, p8):
        pltpu.make_async_copy(hs_hbm.at[sidxs[p8]], rows[p4], gsems[p4]).wait()

    def sstart(p4, p8):
        pltpu.async_copy(rows[p4], acc_sh.at[didxs[p8]], ssems[p4], add=True)

    def swait(p4, p8):
        pltpu.make_async_copy(rows[p4], acc_sh.at[didxs[p8]], ssems[p4]).wait()

    def step(j, p4, p8, do_c, do_d, do_ef):
        gwait(p4, p8)
        sstart(p4, p8)
        if do_c:
            idxstart(j + 4, (p8 + 4) % 8)
        if do_d:
            swait((p4 + 3) % 4, (p8 + 3) % 8)
        if do_ef:
            iwait(j + 3, (p8 + 3) % 8)
            gather((p4 + 3) % 4, (p8 + 3) % 8)

    for j in range(4):
        idxstart(j, j)
    iwait(0, 0)
    gather(0, 0)
    iwait(1, 1)
    gather(1, 1)
    iwait(2, 2)
    gather(2, 2)
    step(0, 0, 0, True, False, True)

    def body(i, carry):
        j0 = 8 * i + 1
        for t in range(8):
            step(j0 + t, (t + 1) % 4, (t + 1) % 8, True, True, True)
        return carry

    NTAIL = NCHUNK - 12
    nbody = (NTAIL - 1) // 8
    lax.fori_loop(0, nbody, body, 0)
    for j in range(NTAIL, NCHUNK):
        step(j, j % 4, j % 8, j + 4 < NCHUNK, True, j + 3 < NCHUNK)
    swait((NCHUNK - 1) % 4, (NCHUNK - 1) % 8)

    plsc.subcore_barrier()
    pltpu.sync_copy(acc_sh.at[pl.ds(s * RPT, RPT)],
                    accp_hbm.at[c, pl.ds(s * RPT, RPT)])


_scat_kernel = pl.kernel(
    _scat_body,
    out_type=jax.ShapeDtypeStruct((NC, NPAD, D), jnp.float32),
    mesh=_mesh,
    scratch_types=(
        [pltpu.VMEM((CHUNK,), jnp.int32)] * 16
        + [pltpu.VMEM((CHUNK, D), jnp.float32)] * 4
        + [pltpu.VMEM((ZCH, D), jnp.float32)]
        + [pltpu.VMEM_SHARED((NPAD, D), jnp.float32)]
        + [pltpu.SemaphoreType.DMA] * 16
    ),
)


def _mm_body(x_ref, w_ref, degp_ref, hs_ref, dis_ref):
    deg = degp_ref[0] + degp_ref[1] + 1.0
    dis = lax.rsqrt(deg)[:N, None]
    h = jnp.dot(x_ref[...], w_ref[...], preferred_element_type=jnp.float32)
    hs_ref[...] = h * dis
    dis_ref[...] = dis


_mm_kernel = pl.pallas_call(
    _mm_body,
    out_shape=[jax.ShapeDtypeStruct((N, D), jnp.float32),
               jax.ShapeDtypeStruct((N, 1), jnp.float32)],
)


def _fuse_mid_body(accp_ref, hs_ref, dis_ref, b_ref, g_ref, be_ref, w2_ref,
                   hs2_ref):
    dis = dis_ref[...]
    acc = accp_ref[0, :N] + accp_ref[1, :N]
    o = (acc + hs_ref[...]) * dis + b_ref[...]
    m = jnp.mean(o, axis=0, keepdims=True)
    cen = o - m
    v = jnp.mean(cen * cen, axis=0, keepdims=True)
    y = g_ref[...] * (cen * lax.rsqrt(v + EPS)) + be_ref[...]
    x2 = jnp.maximum(y, 0.0)
    h2 = jnp.dot(x2, w2_ref[...], preferred_element_type=jnp.float32)
    hs2_ref[...] = h2 * dis


_fuse_mid_kernel = pl.pallas_call(
    _fuse_mid_body,
    out_shape=jax.ShapeDtypeStruct((N, D), jnp.float32),
)


def _fuse_out_body(accp_ref, hs_ref, dis_ref, b_ref, g_ref, be_ref, out_ref):
    dis = dis_ref[...]
    acc = accp_ref[0, :N] + accp_ref[1, :N]
    o = (acc + hs_ref[...]) * dis + b_ref[...]
    m = jnp.mean(o, axis=0, keepdims=True)
    cen = o - m
    v = jnp.mean(cen * cen, axis=0, keepdims=True)
    out_ref[...] = g_ref[...] * (cen * lax.rsqrt(v + EPS)) + be_ref[...]


_fuse_out_kernel = pl.pallas_call(
    _fuse_out_body,
    out_shape=jax.ShapeDtypeStruct((N, D), jnp.float32),
)


@jax.jit
def kernel(x, edge_index, W1, b1, g1, be1, W2, b2, g2, be2):
    ei = edge_index.astype(jnp.int32)
    src1 = ei[0]
    dst1 = ei[1]
    dst3 = dst1.reshape(NW, NCHUNK, CHUNK)

    degp = _deg_kernel(dst3)

    b1r, g1r, be1r = b1[None, :], g1[None, :], be1[None, :]
    b2r, g2r, be2r = b2[None, :], g2[None, :], be2[None, :]

    hs1, dis = _mm_kernel(x, W1, degp)
    acc1 = _scat_kernel(hs1, src1, dst1)
    hs2 = _fuse_mid_kernel(acc1, hs1, dis, b1r, g1r, be1r, W2)
    acc2 = _scat_kernel(hs2, src1, dst1)
    return _fuse_out_kernel(acc2, hs2, dis, b2r, g2r, be2r)
